# baseline (speedup 1.0000x reference)
import functools

import jax
import jax.numpy as jnp
from jax import lax
from jax.experimental import pallas as pl
from jax.experimental.pallas import tpu as pltpu
from jax.experimental.pallas import tpu_sc as plsc

NC = 2
NS = 16
NW = NC * NS


NSTRIP_FULL = 244
V = 1000000


W = 512
JR = W // 4
NROUND = 61


def _repack_strip(tv, ov):
    iota = lax.broadcasted_iota(jnp.int32, (16,), 0)

    def jstep(j, c):
        for k in range(8):
            rvec = iota + 16 * (k % 2)
            cvec = (4 * j + k // 2) + 0 * iota
            vals = plsc.load_gather(tv, [rvec, cvec])
            ov[j, pl.ds(16 * k, 16)] = vals
        return c
    lax.fori_loop(0, JR, jstep, 0, unroll=16)


def _k1_body(table_t, tail_hbm, t4_out, tv0, tv1, ov0, ov1, isem, osem):
    wid = lax.axis_index("s") * NC + lax.axis_index("c")

    def strip_of(j):
        return 32 * j + wid

    def issue_in(s, buf):
        pltpu.async_copy(
            table_t.at[:, pl.ds(pl.multiple_of(W * s, 128), W)], buf, isem)

    def wait_in():
        pltpu.make_async_copy(table_t.at[:, pl.ds(0, W)], tv0, isem).wait()

    def wait_out():
        pltpu.make_async_copy(ov0, t4_out.at[pl.ds(0, JR)], osem).wait()

    issue_in(strip_of(0), tv0)

    def step(j, c):
        p = lax.rem(j, 2)
        wait_in()

        @pl.when(j < NROUND - 1)
        def _():
            @pl.when(p == 0)
            def _():
                issue_in(strip_of(j + 1), tv1)

            @pl.when(p == 1)
            def _():
                issue_in(strip_of(j + 1), tv0)

        @pl.when(j >= 2)
        def _():
            wait_out()

        row0 = pl.multiple_of(JR * strip_of(j), JR)

        @pl.when(p == 0)
        def _():
            _repack_strip(tv0, ov0)
            pltpu.async_copy(ov0, t4_out.at[pl.ds(row0, JR)], osem)

        @pl.when(p == 1)
        def _():
            _repack_strip(tv1, ov1)
            pltpu.async_copy(ov1, t4_out.at[pl.ds(row0, JR)], osem)

        return c

    lax.fori_loop(0, NROUND, step, 0)
    wait_out()
    wait_out()

    @pl.when(wid == 0)
    def _():
        pltpu.sync_copy(table_t.at[:, pl.ds(1952 * W, W)], tv0)
        _repack_strip(tv0, ov0)
        pltpu.sync_copy(ov0, t4_out.at[pl.ds(1952 * JR, JR)])

    @pl.when(wid == 4)
    def _():
        pltpu.sync_copy(tail_hbm, ov0.at[pl.ds(0, 16), :])
        pltpu.sync_copy(ov0.at[pl.ds(0, 16), :],
                        t4_out.at[pl.ds(7812 * 32, 16)])


@jax.jit
def _repack(table_t, tail):
    mesh = plsc.VectorSubcoreMesh(core_axis_name="c", subcore_axis_name="s")
    return pl.kernel(
        _k1_body,
        out_type=jax.ShapeDtypeStruct((V // 4, 128), jnp.float32),
        mesh=mesh,
        scratch_types=[
            pltpu.VMEM((32, W), jnp.float32),
            pltpu.VMEM((32, W), jnp.float32),
            pltpu.VMEM((JR, 128), jnp.float32),
            pltpu.VMEM((JR, 128), jnp.float32),
            pltpu.SemaphoreType.DMA,
            pltpu.SemaphoreType.DMA,
        ],
        compiler_params=pltpu.CompilerParams(use_tc_tiling_on_sc=True, needs_layout_passes=False),
    )(table_t, tail)


BLKB = 128
HP = 25
NBLK = 4
ROWS = 256


def _body(idx_hbm, table4_hbm, out3_hbm,
          idx_all, gidx0, gidx1, rbuf0, rbuf1, rows0, rows1, ov0, ov1,
          gsem, osem):
    wid = lax.axis_index("s") * NC + lax.axis_index("c")
    iota = lax.broadcasted_iota(jnp.int32, (16,), 0)

    def build(i, gidx, rbuf):
        h = 2 * i
        for half in range(2):
            for k in range(8):
                addr = (h + half) + 800 * k + 50 * iota
                v = plsc.load_gather(idx_all, [addr])
                gidx[pl.ds(128 * half + 16 * k, 16)] = v >> 2
                rbuf[pl.ds(128 * half + 16 * k, 16)] = (v & 3) * 32

    def transpose_half(rows_v, rbuf, out_v, half):
        def kstep(k, c):
            base = 128 * half + 16 * k
            rvec = base + iota
            rvals = rbuf[pl.ds(base, 16)]
            for f in range(32):
                vals = plsc.load_gather(rows_v, [rvec, rvals + f])
                out_v[f, pl.ds(16 * k, 16)] = vals
            return c
        lax.fori_loop(0, 8, kstep, 0, unroll=2)

    def block(bi, carry):
        b0 = pl.multiple_of((4 * wid + bi) * BLKB, BLKB)
        pltpu.sync_copy(idx_hbm.at[pl.ds(b0 * 50, 50 * BLKB)], idx_all)

        def wait_gather():
            pltpu.make_async_copy(
                table4_hbm.at[pl.ds(0, ROWS)], rows0, gsem).wait()

        def wait_out():
            pltpu.make_async_copy(
                ov0, out3_hbm.at[0, :, pl.ds(b0, BLKB)], osem).wait()
            pltpu.make_async_copy(
                ov1, out3_hbm.at[0, :, pl.ds(b0, BLKB)], osem).wait()

        def process(rows_v, rbuf, h):
            transpose_half(rows_v, rbuf, ov0, 0)
            pltpu.async_copy(ov0, out3_hbm.at[h, :, pl.ds(b0, BLKB)], osem)
            transpose_half(rows_v, rbuf, ov1, 1)
            pltpu.async_copy(ov1, out3_hbm.at[h + 1, :, pl.ds(b0, BLKB)], osem)

        build(0, gidx0, rbuf0)
        pltpu.async_copy(table4_hbm.at[gidx0], rows0, gsem)

        def unit(i, carry2):
            p = lax.rem(i, 2)

            @pl.when(i < HP - 1)
            def _():
                @pl.when(p == 0)
                def _():
                    build(i + 1, gidx1, rbuf1)
                    pltpu.async_copy(table4_hbm.at[gidx1], rows1, gsem)

                @pl.when(p == 1)
                def _():
                    build(i + 1, gidx0, rbuf0)
                    pltpu.async_copy(table4_hbm.at[gidx0], rows0, gsem)

            wait_gather()

            @pl.when(i >= 1)
            def _():
                wait_out()

            h = 2 * i

            @pl.when(p == 0)
            def _():
                process(rows0, rbuf0, h)

            @pl.when(p == 1)
            def _():
                process(rows1, rbuf1, h)

            return carry2

        lax.fori_loop(0, HP, unit, 0)
        wait_out()
        return carry

    lax.fori_loop(0, NBLK, block, 0)


@functools.partial(jax.jit, static_argnames=("n", "d"))
def _gather(flat_idx, table4, n, d):
    mesh = plsc.VectorSubcoreMesh(core_axis_name="c", subcore_axis_name="s")
    return pl.kernel(
        _body,
        out_type=jax.ShapeDtypeStruct((50, d, n // 50), jnp.float32),
        mesh=mesh,
        scratch_types=[
            pltpu.VMEM((50 * BLKB,), jnp.int32),
            pltpu.VMEM((ROWS,), jnp.int32),
            pltpu.VMEM((ROWS,), jnp.int32),
            pltpu.VMEM((ROWS,), jnp.int32),
            pltpu.VMEM((ROWS,), jnp.int32),
            pltpu.VMEM((ROWS, 128), jnp.float32),
            pltpu.VMEM((ROWS, 128), jnp.float32),
            pltpu.VMEM((32, BLKB), jnp.float32),
            pltpu.VMEM((32, BLKB), jnp.float32),
            pltpu.SemaphoreType.DMA,
            pltpu.SemaphoreType.DMA,
        ],
        compiler_params=pltpu.CompilerParams(use_tc_tiling_on_sc=True, needs_layout_passes=False),
    )(flat_idx, table4)


def kernel(action_idx, table):
    b, h = action_idx.shape
    n = b * h
    d = table.shape[1]
    flat_idx = action_idx.reshape(n).astype(jnp.int32)
    tail = table[V - 64:, :].reshape(16, 128)
    table4 = _repack(jnp.transpose(table), tail)
    out3 = _gather(flat_idx, table4, n, d)
    return jnp.transpose(out3, (2, 0, 1))


# speedup vs baseline: 1.2472x; 1.2472x over previous
import functools

import jax
import jax.numpy as jnp
from jax import lax
from jax.experimental import pallas as pl
from jax.experimental.pallas import tpu as pltpu
from jax.experimental.pallas import tpu_sc as plsc

NC = 2
NS = 16
NW = NC * NS

BLKB = 128
HP = 25
NBLK = 4
ROWS = 256
GBYTES = ROWS * 128 * 4
OBYTES = 2 * 32 * BLKB * 4


def _body(idx_hbm, table4_hbm, out3_hbm,
          idx_all, gidx0, gidx1, rbuf0, rbuf1, rows0, rows1, ov0, ov1,
          gsem, osem):
    wid = lax.axis_index("s") * NC + lax.axis_index("c")
    iota = lax.broadcasted_iota(jnp.int32, (16,), 0)

    def build(i, gidx, rbuf):
        h = 2 * i
        for half in range(2):
            for k in range(8):
                addr = (h + half) + 800 * k + 50 * iota
                v = plsc.load_gather(idx_all, [addr])
                gidx[pl.ds(128 * half + 16 * k, 16)] = v >> 2
                rbuf[pl.ds(128 * half + 16 * k, 16)] = (v & 3) * 32

    def transpose_half(rows_v, rbuf, out_v, half):
        def kstep(k, c):
            base = 128 * half + 16 * k
            rvec = base + iota
            rvals = rbuf[pl.ds(base, 16)]
            for f in range(32):
                vals = plsc.load_gather(rows_v, [rvec, rvals + f])
                out_v[f, pl.ds(16 * k, 16)] = vals
            return c
        lax.fori_loop(0, 8, kstep, 0, unroll=2)

    def process(rows_v, rbuf, h, b0):
        transpose_half(rows_v, rbuf, ov0, 0)
        pltpu.async_copy(ov0, out3_hbm.at[h, :, pl.ds(b0, BLKB)], osem)
        transpose_half(rows_v, rbuf, ov1, 1)
        pltpu.async_copy(ov1, out3_hbm.at[h + 1, :, pl.ds(b0, BLKB)], osem)

    def block(bi, carry):
        b0 = pl.multiple_of((4 * wid + bi) * BLKB, BLKB)
        pltpu.sync_copy(idx_hbm.at[pl.ds(b0 * 50, 50 * BLKB)], idx_all)

        build(0, gidx0, rbuf0)
        pltpu.async_copy(table4_hbm.at[gidx0], rows0, gsem)

        def wait_gather(i):
            pltpu.make_async_copy(
                table4_hbm.at[pl.ds(0, ROWS)], rows0, gsem).wait()

        def wait_out(i):
            pltpu.make_async_copy(
                ov0, out3_hbm.at[0, :, pl.ds(b0, BLKB)], osem).wait()
            pltpu.make_async_copy(
                ov1, out3_hbm.at[0, :, pl.ds(b0, BLKB)], osem).wait()

        def unit(i, carry2):
            p = lax.rem(i, 2)

            @pl.when(i < HP - 1)
            def _():
                @pl.when(p == 0)
                def _():
                    build(i + 1, gidx1, rbuf1)
                    pltpu.async_copy(table4_hbm.at[gidx1], rows1, gsem)

                @pl.when(p == 1)
                def _():
                    build(i + 1, gidx0, rbuf0)
                    pltpu.async_copy(table4_hbm.at[gidx0], rows0, gsem)

            wait_gather(i)

            @pl.when(i >= 1)
            def _():
                wait_out(i)

            h = 2 * i

            @pl.when(p == 0)
            def _():
                process(rows0, rbuf0, h, b0)

            @pl.when(p == 1)
            def _():
                process(rows1, rbuf1, h, b0)

            return carry2

        lax.fori_loop(0, HP, unit, 0)
        wait_out(HP)
        return carry

    lax.fori_loop(0, NBLK, block, 0)


@functools.partial(jax.jit, static_argnames=("n", "d"))
def _gather(flat_idx, table4, n, d):
    mesh = plsc.VectorSubcoreMesh(core_axis_name="c", subcore_axis_name="s")
    return pl.kernel(
        _body,
        out_type=jax.ShapeDtypeStruct((50, d, n // 50), jnp.float32),
        mesh=mesh,
        scratch_types=[
            pltpu.VMEM((50 * BLKB,), jnp.int32),
            pltpu.VMEM((ROWS,), jnp.int32),
            pltpu.VMEM((ROWS,), jnp.int32),
            pltpu.VMEM((ROWS,), jnp.int32),
            pltpu.VMEM((ROWS,), jnp.int32),
            pltpu.VMEM((ROWS, 128), jnp.float32),
            pltpu.VMEM((ROWS, 128), jnp.float32),
            pltpu.VMEM((32, BLKB), jnp.float32),
            pltpu.VMEM((32, BLKB), jnp.float32),
            pltpu.SemaphoreType.DMA,
            pltpu.SemaphoreType.DMA,
        ],
        compiler_params=pltpu.CompilerParams(use_tc_tiling_on_sc=True, needs_layout_passes=False),
    )(flat_idx, table4)


def kernel(action_idx, table):
    b, h = action_idx.shape
    n = b * h
    d = table.shape[1]
    flat_idx = action_idx.reshape(n).astype(jnp.int32)
    table4 = table.reshape(table.shape[0] // 4, 128)
    out3 = _gather(flat_idx, table4, n, d)
    return jnp.transpose(out3, (2, 0, 1))
